# baseline (speedup 1.0000x reference)
import functools

import jax
import jax.numpy as jnp
from jax import lax
from jax.experimental import pallas as pl
from jax.experimental.pallas import tpu as pltpu
from jax.experimental.pallas import tpu_sc as plsc

NUM_WORKERS = 32
CHUNK = 2048
LANES = 16
NBUF = 2


def _make_sc_call(n, nx, ny, nz):
    n_chunks = -(-n // CHUNK)
    total_chunks = -(-n_chunks // (2 * NUM_WORKERS)) * (2 * NUM_WORKERS)
    cpw = total_chunks // NUM_WORKERS
    last_base = n - CHUNK
    sx = ny * nz
    sy = nz

    mesh = plsc.VectorSubcoreMesh(core_axis_name="c", subcore_axis_name="s")

    bank_scratch = (
        [pltpu.VMEM((3 * CHUNK,), jnp.float32)]
        + [pltpu.VMEM((CHUNK,), jnp.float32) for _ in range(3)]
        + [pltpu.VMEM((CHUNK,), jnp.int32) for _ in range(8)]
        + [pltpu.VMEM((CHUNK,), jnp.float32) for _ in range(16)]
        + [pltpu.SemaphoreType.DMA]
    )
    scratch = (
        bank_scratch * NBUF
        + [pltpu.VMEM((CHUNK,), jnp.float32) for _ in range(2)]
        + [pltpu.VMEM((LANES,), jnp.float32) for _ in range(6)]
    )

    @functools.partial(
        pl.kernel,
        mesh=mesh,
        out_type=(
            jax.ShapeDtypeStruct((n,), jnp.float32),
            jax.ShapeDtypeStruct((n,), jnp.float32),
        ),
        scratch_types=scratch,
    )
    def sc_call(pos_h, par_h, tab_h, outm_h, outs_h, *refs):
        g0_h = tab_h.at[pl.ds(0, sx * nx)]
        g1_h = tab_h.at[pl.ds(sx * nx, sx * nx)]
        nb = 29
        banks = []
        for b in range(NBUF):
            r = refs[b * nb:(b + 1) * nb]
            banks.append(dict(pos=r[0], frac=r[1:4], idx=r[4:12],
                              res=r[12:28], sem=r[28]))
        out_v = refs[2 * nb:2 * nb + 2]
        par_v = refs[2 * nb + 2:2 * nb + 8]

        wid = lax.axis_index("s") * 2 + lax.axis_index("c")

        for d in range(6):
            pltpu.sync_copy(par_h.at[pl.ds(d * LANES, LANES)], par_v[d])
        minx = par_v[0][:]
        miny = par_v[1][:]
        minz = par_v[2][:]
        sclx = par_v[3][:]
        scly = par_v[4][:]
        sclz = par_v[5][:]

        lane = lax.iota(jnp.int32, LANES)
        deint = []
        for d in range(3):
            slot = 3 * lane + d
            deint.append(dict(
                m0=slot < 16,
                m1=slot < 32,
                p0=jnp.clip(slot, 0, 15),
                p1=jnp.clip(slot - 16, 0, 15),
                p2=jnp.clip(slot - 32, 0, 15),
            ))

        def chunk_base(j):
            t = j * NUM_WORKERS + wid
            return jnp.minimum(t * CHUNK, last_base)

        def load_and_index(j, bk):
            base = chunk_base(j)
            pltpu.sync_copy(pos_h.at[pl.ds(base * 3, 3 * CHUNK)], bk["pos"])

            def index_body(g, c):
                sl = pl.ds(g * LANES, LANES)
                v0 = bk["pos"][pl.ds(g * 48, LANES)]
                v1 = bk["pos"][pl.ds(g * 48 + 16, LANES)]
                v2 = bk["pos"][pl.ds(g * 48 + 32, LANES)]

                def coord(d):
                    t = deint[d]
                    return jnp.where(
                        t["m0"], jnp.take(v0, t["p0"]),
                        jnp.where(t["m1"], jnp.take(v1, t["p1"]),
                                  jnp.take(v2, t["p2"])))

                fx = jnp.maximum((coord(0) - minx) * sclx, 0.0)
                fy = jnp.maximum((coord(1) - miny) * scly, 0.0)
                fz = jnp.maximum((coord(2) - minz) * sclz, 0.0)
                x0 = jnp.minimum(fx.astype(jnp.int32), nx - 2)
                y0 = jnp.minimum(fy.astype(jnp.int32), ny - 2)
                z0 = jnp.minimum(fz.astype(jnp.int32), nz - 2)
                bk["frac"][0][sl] = fx - x0.astype(jnp.float32)
                bk["frac"][1][sl] = fy - y0.astype(jnp.float32)
                bk["frac"][2][sl] = fz - z0.astype(jnp.float32)
                b = x0 * sx + y0 * sy + z0
                bk["idx"][0][sl] = b
                bk["idx"][1][sl] = b + 1
                bk["idx"][2][sl] = b + sy
                bk["idx"][3][sl] = b + (sy + 1)
                bk["idx"][4][sl] = b + sx
                bk["idx"][5][sl] = b + (sx + 1)
                bk["idx"][6][sl] = b + (sx + sy)
                bk["idx"][7][sl] = b + (sx + sy + 1)
                return c

            lax.fori_loop(0, CHUNK // LANES, index_body, 0)

        def gathers(bk):
            return (
                [pltpu.make_async_copy(g0_h.at[bk["idx"][k]], bk["res"][k],
                                       bk["sem"]) for k in range(8)]
                + [pltpu.make_async_copy(g1_h.at[bk["idx"][k]],
                                         bk["res"][8 + k], bk["sem"])
                   for k in range(8)]
            )

        def fire(bk):
            for cp in gathers(bk):
                cp.start()

        def drain(bk):
            for cp in gathers(bk):
                cp.wait()

        def combine_store(j, bk):
            base = chunk_base(j)
            res_v = bk["res"]

            def combine_body(g, c):
                sl = pl.ds(g * LANES, LANES)
                tx = bk["frac"][0][sl]
                ty = bk["frac"][1][sl]
                tz = bk["frac"][2][sl]
                ux = 1.0 - tx
                uy = 1.0 - ty
                uz = 1.0 - tz
                c00 = uy * uz
                c01 = uy * tz
                c10 = ty * uz
                c11 = ty * tz
                w0 = ux * c00
                w1 = ux * c01
                w2 = ux * c10
                w3 = ux * c11
                w4 = tx * c00
                w5 = tx * c01
                w6 = tx * c10
                w7 = tx * c11
                m = (w0 * res_v[0][sl] + w1 * res_v[1][sl]
                     + w2 * res_v[2][sl] + w3 * res_v[3][sl]
                     + w4 * res_v[4][sl] + w5 * res_v[5][sl]
                     + w6 * res_v[6][sl] + w7 * res_v[7][sl])
                s = (w0 * res_v[8][sl] + w1 * res_v[9][sl]
                     + w2 * res_v[10][sl] + w3 * res_v[11][sl]
                     + w4 * res_v[12][sl] + w5 * res_v[13][sl]
                     + w6 * res_v[14][sl] + w7 * res_v[15][sl])
                out_v[0][sl] = m
                out_v[1][sl] = s
                return c

            lax.fori_loop(0, CHUNK // LANES, combine_body, 0)
            pltpu.sync_copy(out_v[0], outm_h.at[pl.ds(base, CHUNK)])
            pltpu.sync_copy(out_v[1], outs_h.at[pl.ds(base, CHUNK)])

        load_and_index(0, banks[0])
        fire(banks[0])

        def pair_body(p, carry):
            j0 = 2 * p
            j1 = j0 + 1
            j2 = j0 + 2
            load_and_index(j1, banks[1])
            fire(banks[1])
            drain(banks[0])
            combine_store(j0, banks[0])

            @pl.when(j2 < cpw)
            def _():
                load_and_index(j2, banks[0])
                fire(banks[0])

            drain(banks[1])
            combine_store(j1, banks[1])
            return carry

        lax.fori_loop(0, cpw // 2, pair_body, 0)

    return sc_call


def kernel(pos, grid, min_bound, max_bound):
    n = pos.shape[0]
    _, nx, ny, nz = grid.shape

    n_al = -(-n // 8) * 8
    if n_al != n:
        pos = jnp.concatenate([pos, pos[: n_al - n]])

    grid_range = jnp.clip(max_bound - min_bound, 1e-6, None)
    dims = jnp.array([nx - 1, ny - 1, nz - 1], dtype=jnp.float32)
    scales = dims / grid_range
    params = jnp.concatenate(
        [
            jnp.repeat(min_bound.astype(jnp.float32), LANES),
            jnp.repeat(scales.astype(jnp.float32), LANES),
        ]
    )

    table = grid.reshape(-1)

    sc_call = _make_sc_call(n_al, nx, ny, nz)
    outm, outs = sc_call(pos.reshape(-1), params, table)
    if n_al != n:
        return (outm[:n], outs[:n])
    return (outm, outs)


# speedup vs baseline: 5.2327x; 5.2327x over previous
import functools

import jax
import jax.numpy as jnp
from jax import lax
from jax.experimental import pallas as pl
from jax.experimental.pallas import tpu as pltpu
from jax.experimental.pallas import tpu_sc as plsc

NUM_WORKERS = 32
CHUNK = 2048
LANES = 16
NBUF = 2


def _make_sc_call(n, nx, ny, nz):
    n_chunks = -(-n // CHUNK)
    total_chunks = -(-n_chunks // (2 * NUM_WORKERS)) * (2 * NUM_WORKERS)
    cpw = total_chunks // NUM_WORKERS
    last_base = n - CHUNK
    sx = ny * nz
    sy = nz

    mesh = plsc.VectorSubcoreMesh(core_axis_name="c", subcore_axis_name="s")

    bank_scratch = (
        [pltpu.VMEM((CHUNK,), jnp.float32) for _ in range(3)]
        + [pltpu.VMEM((CHUNK,), jnp.float32) for _ in range(3)]
        + [pltpu.VMEM((CHUNK,), jnp.int32) for _ in range(8)]
        + [pltpu.VMEM((CHUNK,), jnp.float32) for _ in range(16)]
        + [pltpu.SemaphoreType.DMA]
    )
    scratch = (
        bank_scratch * NBUF
        + [pltpu.VMEM((CHUNK,), jnp.float32) for _ in range(2)]
        + [pltpu.VMEM((LANES,), jnp.float32) for _ in range(6)]
    )

    @functools.partial(
        pl.kernel,
        mesh=mesh,
        out_type=(
            jax.ShapeDtypeStruct((n,), jnp.float32),
            jax.ShapeDtypeStruct((n,), jnp.float32),
        ),
        scratch_types=scratch,
    )
    def sc_call(posx_h, posy_h, posz_h, par_h, tab_h,
                outm_h, outs_h, *refs):
        g0_h = tab_h.at[pl.ds(0, sx * nx)]
        g1_h = tab_h.at[pl.ds(sx * nx, sx * nx)]
        nb = 31
        banks = []
        for b in range(NBUF):
            r = refs[b * nb:(b + 1) * nb]
            banks.append(dict(pos=r[0:3], frac=r[3:6], idx=r[6:14],
                              res=r[14:30], sem=r[30]))
        out_v = refs[2 * nb:2 * nb + 2]
        par_v = refs[2 * nb + 2:2 * nb + 8]

        wid = lax.axis_index("s") * 2 + lax.axis_index("c")

        for d in range(6):
            pltpu.sync_copy(par_h.at[pl.ds(d * LANES, LANES)], par_v[d])
        minx = par_v[0][:]
        miny = par_v[1][:]
        minz = par_v[2][:]
        sclx = par_v[3][:]
        scly = par_v[4][:]
        sclz = par_v[5][:]

        def chunk_base(j):
            t = j * NUM_WORKERS + wid
            return jnp.minimum(t * CHUNK, last_base)

        def load_and_index(j, bk):
            base = chunk_base(j)
            pltpu.sync_copy(posx_h.at[pl.ds(base, CHUNK)], bk["pos"][0])
            pltpu.sync_copy(posy_h.at[pl.ds(base, CHUNK)], bk["pos"][1])
            pltpu.sync_copy(posz_h.at[pl.ds(base, CHUNK)], bk["pos"][2])

            def index_body(g, c):
                sl = pl.ds(g * LANES, LANES)
                fx = jnp.maximum((bk["pos"][0][sl] - minx) * sclx, 0.0)
                fy = jnp.maximum((bk["pos"][1][sl] - miny) * scly, 0.0)
                fz = jnp.maximum((bk["pos"][2][sl] - minz) * sclz, 0.0)
                x0 = jnp.minimum(fx.astype(jnp.int32), nx - 2)
                y0 = jnp.minimum(fy.astype(jnp.int32), ny - 2)
                z0 = jnp.minimum(fz.astype(jnp.int32), nz - 2)
                bk["frac"][0][sl] = fx - x0.astype(jnp.float32)
                bk["frac"][1][sl] = fy - y0.astype(jnp.float32)
                bk["frac"][2][sl] = fz - z0.astype(jnp.float32)
                b = x0 * sx + y0 * sy + z0
                bk["idx"][0][sl] = b
                bk["idx"][1][sl] = b + 1
                bk["idx"][2][sl] = b + sy
                bk["idx"][3][sl] = b + (sy + 1)
                bk["idx"][4][sl] = b + sx
                bk["idx"][5][sl] = b + (sx + 1)
                bk["idx"][6][sl] = b + (sx + sy)
                bk["idx"][7][sl] = b + (sx + sy + 1)
                return c

            lax.fori_loop(0, CHUNK // LANES, index_body, 0)

        def gathers(bk):
            return (
                [pltpu.make_async_copy(g0_h.at[bk["idx"][k]], bk["res"][k],
                                       bk["sem"]) for k in range(8)]
                + [pltpu.make_async_copy(g1_h.at[bk["idx"][k]],
                                         bk["res"][8 + k], bk["sem"])
                   for k in range(8)]
            )

        def fire(bk):
            for cp in gathers(bk):
                cp.start()

        def drain(bk):
            for cp in gathers(bk):
                cp.wait()

        def combine_store(j, bk):
            base = chunk_base(j)
            res_v = bk["res"]

            def combine_body(g, c):
                sl = pl.ds(g * LANES, LANES)
                tx = bk["frac"][0][sl]
                ty = bk["frac"][1][sl]
                tz = bk["frac"][2][sl]
                ux = 1.0 - tx
                uy = 1.0 - ty
                uz = 1.0 - tz
                c00 = uy * uz
                c01 = uy * tz
                c10 = ty * uz
                c11 = ty * tz
                w0 = ux * c00
                w1 = ux * c01
                w2 = ux * c10
                w3 = ux * c11
                w4 = tx * c00
                w5 = tx * c01
                w6 = tx * c10
                w7 = tx * c11
                m = (w0 * res_v[0][sl] + w1 * res_v[1][sl]
                     + w2 * res_v[2][sl] + w3 * res_v[3][sl]
                     + w4 * res_v[4][sl] + w5 * res_v[5][sl]
                     + w6 * res_v[6][sl] + w7 * res_v[7][sl])
                s = (w0 * res_v[8][sl] + w1 * res_v[9][sl]
                     + w2 * res_v[10][sl] + w3 * res_v[11][sl]
                     + w4 * res_v[12][sl] + w5 * res_v[13][sl]
                     + w6 * res_v[14][sl] + w7 * res_v[15][sl])
                out_v[0][sl] = m
                out_v[1][sl] = s
                return c

            lax.fori_loop(0, CHUNK // LANES, combine_body, 0)
            pltpu.sync_copy(out_v[0], outm_h.at[pl.ds(base, CHUNK)])
            pltpu.sync_copy(out_v[1], outs_h.at[pl.ds(base, CHUNK)])

        load_and_index(0, banks[0])
        fire(banks[0])

        def pair_body(p, carry):
            j0 = 2 * p
            j1 = j0 + 1
            j2 = j0 + 2
            load_and_index(j1, banks[1])
            fire(banks[1])
            drain(banks[0])
            combine_store(j0, banks[0])

            @pl.when(j2 < cpw)
            def _():
                load_and_index(j2, banks[0])
                fire(banks[0])

            drain(banks[1])
            combine_store(j1, banks[1])
            return carry

        lax.fori_loop(0, cpw // 2, pair_body, 0)

    return sc_call


def kernel(pos, grid, min_bound, max_bound):
    n = pos.shape[0]
    _, nx, ny, nz = grid.shape

    n_al = -(-n // 8) * 8
    if n_al != n:
        pos = jnp.concatenate([pos, pos[: n_al - n]])

    posx = pos[:, 0]
    posy = pos[:, 1]
    posz = pos[:, 2]

    grid_range = jnp.clip(max_bound - min_bound, 1e-6, None)
    dims = jnp.array([nx - 1, ny - 1, nz - 1], dtype=jnp.float32)
    scales = dims / grid_range
    params = jnp.concatenate(
        [
            jnp.repeat(min_bound.astype(jnp.float32), LANES),
            jnp.repeat(scales.astype(jnp.float32), LANES),
        ]
    )

    table = grid.reshape(-1)

    sc_call = _make_sc_call(n_al, nx, ny, nz)
    outm, outs = sc_call(posx, posy, posz, params, table)
    if n_al != n:
        return (outm[:n], outs[:n])
    return (outm, outs)
